# baseline (speedup 1.0000x reference)
import functools

import jax
import jax.numpy as jnp
from jax import lax
from jax.experimental import pallas as pl
from jax.experimental.pallas import tpu as pltpu

_EPS = 1e-06
_D = 1024
_NMOD = 64
_SEQ = 8192
_BLK = 256
_NBLK = _SEQ // _BLK
_WAYS = 4
_NB = _NBLK // _WAYS


def _fused_body(*refs):
    x_refs = refs[:_WAYS]
    keep_ref = refs[_WAYS]
    w0_ref, w1_ref, b_ref = refs[_WAYS + 1:_WAYS + 4]
    l0_ref, l1_ref, s0_ref, s1_ref = refs[_WAYS + 4:_WAYS + 8]
    acc_ref, kacc_ref = refs[_WAYS + 8:]

    b_ = pl.program_id(0)
    k = pl.program_id(1)

    dn = (((1,), (0,)), ((), ()))
    part = jnp.zeros((1, _D), jnp.float32)
    ksum = jnp.zeros((1, _BLK), jnp.float32)
    for i in range(_WAYS):
        blk = b_ * _NBLK + i * _NB + k
        keep = keep_ref[blk]
        part = part + lax.dot_general(keep, x_refs[i][0], dn,
                                      preferred_element_type=jnp.float32)
        ksum = ksum + keep

    @pl.when(k == 0)
    def _init():
        acc_ref[...] = part
        kacc_ref[...] = ksum

    @pl.when(k > 0)
    def _accum():
        acc_ref[...] += part
        kacc_ref[...] += ksum

    @pl.when(k == _NB - 1)
    def _finish():
        total = acc_ref[...]
        feats = total / (jnp.sum(kacc_ref[...]) + _EPS)
        dn2 = (((1,), (1,)), ((), ()))
        l0 = lax.dot_general(feats, w0_ref[...], dn2,
                             preferred_element_type=jnp.float32) + b_ref[0, :_NMOD]
        l1 = lax.dot_general(feats, w1_ref[...], dn2,
                             preferred_element_type=jnp.float32) + b_ref[0, _NMOD:]
        l0_ref[0] = l0
        l1_ref[0] = l1
        iota = lax.broadcasted_iota(jnp.int32, (1, _NMOD), 1)
        m0 = jnp.max(l0, axis=1, keepdims=True)
        m1 = jnp.max(l1, axis=1, keepdims=True)
        s0_ref[0] = jnp.min(jnp.where(l0 >= m0, iota, _NMOD), axis=1,
                            keepdims=True)
        s1_ref[0] = jnp.min(jnp.where(l1 >= m1, iota, _NMOD), axis=1,
                            keepdims=True)


def _x_spec(i):
    return pl.BlockSpec((1, _BLK, _D), lambda b_, k, i=i: (b_, i * _NB + k, 0))


def _keep_spec():
    return pl.BlockSpec((4 * _NBLK, 1, _BLK), lambda b_, k: (0, 0, 0))


@jax.jit
def _fused(x, keep, w0, w1, b):
    bsz = x.shape[0]
    grid = (bsz, _NB)
    out = pl.pallas_call(
        _fused_body,
        grid=grid,
        in_specs=(
            [_x_spec(i) for i in range(_WAYS)]
            + [_keep_spec()]
            + [
                pl.BlockSpec((_NMOD, _D), lambda b_, k: (0, 0)),
                pl.BlockSpec((_NMOD, _D), lambda b_, k: (0, 0)),
                pl.BlockSpec((1, 2 * _NMOD), lambda b_, k: (0, 0)),
            ]
        ),
        out_specs=[
            pl.BlockSpec((1, 1, _NMOD), lambda b_, k: (b_, 0, 0)),
            pl.BlockSpec((1, 1, _NMOD), lambda b_, k: (b_, 0, 0)),
            pl.BlockSpec((1, 1, 1), lambda b_, k: (b_, 0, 0)),
            pl.BlockSpec((1, 1, 1), lambda b_, k: (b_, 0, 0)),
        ],
        out_shape=[
            jax.ShapeDtypeStruct((bsz, 1, _NMOD), jnp.float32),
            jax.ShapeDtypeStruct((bsz, 1, _NMOD), jnp.float32),
            jax.ShapeDtypeStruct((bsz, 1, 1), jnp.int32),
            jax.ShapeDtypeStruct((bsz, 1, 1), jnp.int32),
        ],
        scratch_shapes=[
            pltpu.VMEM((1, _D), jnp.float32),
            pltpu.VMEM((1, _BLK), jnp.float32),
        ],
    )(*([x] * _WAYS + [keep, w0, w1, b]))
    return out


def kernel(x, padding_mask, W_out, b_out):
    bsz = x.shape[0]
    x = x.reshape(bsz, _SEQ, _D)
    keep = 1.0 - padding_mask.reshape(bsz * _NBLK, 1, _BLK).astype(jnp.float32)
    w0 = W_out[:_NMOD]
    w1 = W_out[_NMOD:]
    b = b_out.reshape(1, 2 * _NMOD)
    l0, l1, s0, s1 = _fused(x, keep, w0, w1, b)
    logits = jnp.concatenate([l0, l1], axis=1)
    selection = jnp.concatenate([s0[:, :, 0], s1[:, :, 0]], axis=1)
    return (logits, selection, selection)


# speedup vs baseline: 1.1200x; 1.1200x over previous
import jax
import jax.numpy as jnp
from jax import lax
from jax.experimental import pallas as pl
from jax.experimental.pallas import tpu as pltpu

_EPS = 1e-06
_D = 1024
_NMOD = 64
_SEQ = 8192
_BSZ = 4
_ROWS = _BSZ * _SEQ
_CH = 512
_NCHUNK = _ROWS // _CH
_NCB = _SEQ // _CH
_NBUF = 6


def _body(x_hbm, keep_ref, w0_ref, w1_ref, b_ref,
          l0_ref, l1_ref, s0_ref, s1_ref,
          bufs, acc_ref, kacc_ref, sems):
    def start(g, slot):
        pltpu.make_async_copy(
            x_hbm.at[pl.ds(g * _CH, _CH), :], bufs.at[slot], sems.at[slot]
        ).start()

    def wait(slot):
        pltpu.make_async_copy(
            x_hbm.at[pl.ds(0, _CH), :], bufs.at[slot], sems.at[slot]
        ).wait()

    def accum(g, slot):
        keep = keep_ref[g]
        dn = (((1,), (0,)), ((), ()))
        part = lax.dot_general(keep, bufs[slot], dn,
                               preferred_element_type=jnp.float32)
        b_ = g // _NCB
        acc_ref[pl.ds(b_, 1), :] = acc_ref[pl.ds(b_, 1), :] + part
        kacc_ref[pl.ds(b_, 1), :] = kacc_ref[pl.ds(b_, 1), :] + keep

    acc_ref[...] = jnp.zeros((_BSZ, _D), jnp.float32)
    kacc_ref[...] = jnp.zeros((_BSZ, _CH), jnp.float32)

    for j in range(_NBUF):
        start(j, j)

    def step(g, carry):
        slot = lax.rem(g, _NBUF)
        wait(slot)
        accum(g, slot)
        start(g + _NBUF, slot)
        return carry

    lax.fori_loop(0, _NCHUNK - _NBUF, step, 0, unroll=False)

    for g in range(_NCHUNK - _NBUF, _NCHUNK):
        slot = g % _NBUF
        wait(slot)
        accum(g, slot)

    counts = jnp.sum(kacc_ref[...], axis=1, keepdims=True)
    feats = acc_ref[...] / (counts + _EPS)
    dn2 = (((1,), (1,)), ((), ()))
    l0 = lax.dot_general(feats, w0_ref[...], dn2,
                         preferred_element_type=jnp.float32) + b_ref[0, :_NMOD]
    l1 = lax.dot_general(feats, w1_ref[...], dn2,
                         preferred_element_type=jnp.float32) + b_ref[0, _NMOD:]
    l0_ref[...] = l0
    l1_ref[...] = l1
    iota = lax.broadcasted_iota(jnp.int32, (_BSZ, _NMOD), 1)
    m0 = jnp.max(l0, axis=1, keepdims=True)
    m1 = jnp.max(l1, axis=1, keepdims=True)
    s0_ref[...] = jnp.min(jnp.where(l0 >= m0, iota, _NMOD), axis=1,
                          keepdims=True)
    s1_ref[...] = jnp.min(jnp.where(l1 >= m1, iota, _NMOD), axis=1,
                          keepdims=True)


@jax.jit
def _fused(x, keep, w0, w1, b):
    out = pl.pallas_call(
        _body,
        in_specs=[
            pl.BlockSpec(memory_space=pl.ANY),
            pl.BlockSpec(memory_space=pltpu.VMEM),
            pl.BlockSpec(memory_space=pltpu.VMEM),
            pl.BlockSpec(memory_space=pltpu.VMEM),
            pl.BlockSpec(memory_space=pltpu.VMEM),
        ],
        out_specs=[
            pl.BlockSpec(memory_space=pltpu.VMEM),
            pl.BlockSpec(memory_space=pltpu.VMEM),
            pl.BlockSpec(memory_space=pltpu.VMEM),
            pl.BlockSpec(memory_space=pltpu.VMEM),
        ],
        out_shape=[
            jax.ShapeDtypeStruct((_BSZ, _NMOD), jnp.float32),
            jax.ShapeDtypeStruct((_BSZ, _NMOD), jnp.float32),
            jax.ShapeDtypeStruct((_BSZ, 1), jnp.int32),
            jax.ShapeDtypeStruct((_BSZ, 1), jnp.int32),
        ],
        scratch_shapes=[
            pltpu.VMEM((_NBUF, _CH, _D), jnp.float32),
            pltpu.VMEM((_BSZ, _D), jnp.float32),
            pltpu.VMEM((_BSZ, _CH), jnp.float32),
            pltpu.SemaphoreType.DMA((_NBUF,)),
        ],
    )(x, keep, w0, w1, b)
    return out


def kernel(x, padding_mask, W_out, b_out):
    bsz = x.shape[0]
    x = x.reshape(bsz * _SEQ, _D)
    keep = 1.0 - padding_mask.reshape(_NCHUNK, 1, _CH).astype(jnp.float32)
    w0 = W_out[:_NMOD]
    w1 = W_out[_NMOD:]
    b = b_out.reshape(1, 2 * _NMOD)
    l0, l1, s0, s1 = _fused(x, keep, w0, w1, b)
    logits = jnp.concatenate([l0[:, None, :], l1[:, None, :]], axis=1)
    selection = jnp.concatenate([s0, s1], axis=1)
    return (logits, selection, selection)
